# baseline (speedup 1.0000x reference)
import functools
import math

import jax
import jax.numpy as jnp
from jax import lax
from jax.experimental import pallas as pl
from jax.experimental.pallas import tpu as pltpu
from jax.experimental.pallas import tpu_sc as plsc

_EMB = 32
_SCALE = math.sqrt(_EMB)
_NC = 2
_NS = 16
_NW = _NC * _NS
_L = 16
_CH = 128


@functools.lru_cache(maxsize=None)
def _make_lookup(batch: int):
    bpw = batch // _NW
    nblk = bpw // _CH
    mesh = plsc.VectorSubcoreMesh(
        core_axis_name="c", subcore_axis_name="s",
        num_cores=_NC, num_subcores=_NS)

    @functools.partial(
        pl.kernel,
        out_type=jax.ShapeDtypeStruct((batch, _EMB), jnp.float32),
        mesh=mesh,
        compiler_params=pltpu.CompilerParams(use_tc_tiling_on_sc=True),
        scratch_types=[
            pltpu.VMEM((bpw + _L,), jnp.int32),
            pltpu.VMEM((nblk, _CH), jnp.int32),
            [pltpu.VMEM((_CH, 128), jnp.float32)] * 2,
            [pltpu.VMEM((_CH, _EMB), jnp.float32)] * 2,
            [pltpu.SemaphoreType.DMA] * 2,
            [pltpu.SemaphoreType.DMA] * 2,
        ],
    )
    def lookup(tok_hbm, wide_hbm, out_hbm,
               idx_v, row_v, gbufs, obufs, gsems, osems):
        wid = lax.axis_index("s") * _NC + lax.axis_index("c")
        base = wid * bpw
        pltpu.sync_copy(tok_hbm.at[pl.ds(base, bpw)], idx_v.at[pl.ds(0, bpw)])

        @pl.loop(0, nblk)
        def _prep(j):
            for k in range(_CH // _L):
                tok = idx_v[pl.ds(j * _CH + k * _L, _L)]
                row_v[j, pl.ds(k * _L, _L)] = lax.shift_right_logical(tok, 2)

        for p in range(2):
            pltpu.async_copy(wide_hbm.at[row_v.at[p]], gbufs[p], gsems[p])

        @pl.loop(0, nblk, step=2)
        def _grp(g):
            for p in range(2):
                j = g + p
                gbuf, obuf = gbufs[p], obufs[p]
                gs, osm = gsems[p], osems[p]


                pltpu.make_async_copy(
                    wide_hbm.at[row_v.at[p]], gbuf, gs).wait()

                @pl.when(j >= 2)
                def _():
                    pltpu.make_async_copy(
                        obuf, out_hbm.at[pl.ds(0, _CH)], osm).wait()

                @pl.loop(0, _CH)
                def _ext(b):
                    tokv = idx_v[pl.ds(j * _CH + b, _L)]
                    off = lax.shift_left(lax.bitwise_and(tokv[0], 3), 5)
                    for h2 in range(_EMB // _L):
                        v = gbuf[b, pl.ds(off + h2 * _L, _L)]
                        obuf[b, pl.ds(h2 * _L, _L)] = v * _SCALE

                dst = out_hbm.at[
                    pl.ds(pl.multiple_of(base + j * _CH, 8), _CH)]
                pltpu.async_copy(obuf, dst, osm)

                @pl.when(j + 2 < nblk)
                def _():
                    pltpu.async_copy(
                        wide_hbm.at[row_v.at[j + 2]], gbuf, gs)

        for p in range(2):
            pltpu.make_async_copy(
                obufs[p], out_hbm.at[pl.ds(0, _CH)], osems[p]).wait()

    return lookup


def kernel(tokens, table):
    batch, hist = tokens.shape
    b = batch * hist
    tok = tokens.astype(jnp.int32).reshape(b)
    wide = table.reshape(250000, 128)
    out = _make_lookup(b)(tok, wide)
    return out.reshape(batch, hist, _EMB)


# speedup vs baseline: 1.3118x; 1.3118x over previous
import functools
import math

import jax
import jax.numpy as jnp
from jax import lax
from jax.experimental import pallas as pl
from jax.experimental.pallas import tpu as pltpu
from jax.experimental.pallas import tpu_sc as plsc

_EMB = 32
_SCALE = math.sqrt(_EMB)

_NC = 2
_NS = 16
_NW = _NC * _NS
_LANES = 16

_CH = 128


@functools.lru_cache(maxsize=None)
def _make_lookup(batch: int):
    bpw = batch // _NW
    nchunk = bpw // _CH
    mesh = plsc.VectorSubcoreMesh(
        core_axis_name="c", subcore_axis_name="s",
        num_cores=_NC, num_subcores=_NS)

    @functools.partial(
        pl.kernel,
        out_type=jax.ShapeDtypeStruct((batch, _EMB), jnp.float32),
        mesh=mesh,
        compiler_params=pltpu.CompilerParams(use_tc_tiling_on_sc=False),
        scratch_types=[
            pltpu.VMEM((nchunk, _CH), jnp.int32),
            pltpu.VMEM((_CH, _EMB), jnp.float32),
            pltpu.VMEM((_CH, _EMB), jnp.float32),
            pltpu.SemaphoreType.DMA,
            pltpu.SemaphoreType.DMA,
            pltpu.SemaphoreType.DMA,
            pltpu.SemaphoreType.DMA,
        ],
    )
    def lookup(tokens_hbm, table_hbm, out_hbm,
               idx_v, rows0, rows1, gsem0, gsem1, osem0, osem1):
        wid = lax.axis_index("s") * _NC + lax.axis_index("c")
        base = wid * bpw
        pltpu.sync_copy(tokens_hbm.at[wid], idx_v)

        bufs = (rows0, rows1)
        gsems = (gsem0, gsem1)
        osems = (osem0, osem1)

        for b in range(2):
            pltpu.async_copy(table_hbm.at[idx_v.at[b]], bufs[b], gsems[b])

        @pl.loop(0, nchunk, step=2)
        def _grp(g):
            for b in range(2):
                j = g + b
                buf, gs, osm = bufs[b], gsems[b], osems[b]
                pltpu.make_async_copy(table_hbm.at[idx_v.at[b]], buf, gs).wait()

                @pl.loop(0, _CH, unroll=8)
                def _scale(r):
                    for h in range(_EMB // _LANES):
                        sl = pl.ds(h * _LANES, _LANES)
                        buf[r, sl] = buf[r, sl] * _SCALE

                dst = out_hbm.at[pl.ds(base + j * _CH, _CH)]
                pltpu.async_copy(buf, dst, osm)
                pltpu.make_async_copy(buf, dst, osm).wait()

                @pl.when(j + 2 < nchunk)
                def _():
                    pltpu.async_copy(
                        table_hbm.at[idx_v.at[j + 2]], buf, gs)

    return lookup


def kernel(tokens, table):
    batch, hist = tokens.shape
    b = batch * hist
    idx = tokens.astype(jnp.int32).reshape(_NW, b // (_NW * _CH), _CH)
    out = _make_lookup(b)(idx, table)
    return out.reshape(batch, hist, _EMB)
